# baseline (speedup 1.0000x reference)
import functools

import jax
import jax.numpy as jnp
from jax import lax
from jax.experimental import pallas as pl
from jax.experimental.pallas import tpu as pltpu
from jax.experimental.pallas import tpu_sc as plsc

N_NODES = 10000
N_EDGES = 160000
N_ITERS = 7
DH = 10
DE = 11
DIN = 9
MLP_H = 96

NP = 10016
SINK = N_NODES
NW = 32
EPW = 5120
E_PAD = NW * EPW
RPT = NP // 16

@functools.cache
def _mesh():
    return plsc.VectorSubcoreMesh(core_axis_name="c", subcore_axis_name="s")


def _gather_body(state_hbm, src_hbm, dst_hbm, xs_hbm, xd_hbm, idx_v, rows_v, sem):
    wid = lax.axis_index("s") * 2 + lax.axis_index("c")
    base = wid * EPW
    pltpu.sync_copy(src_hbm.at[wid], idx_v)
    pltpu.async_copy(state_hbm.at[idx_v], rows_v, sem).wait()
    pltpu.sync_copy(rows_v, xs_hbm.at[pl.ds(base, EPW)])
    pltpu.sync_copy(dst_hbm.at[wid], idx_v)
    pltpu.async_copy(state_hbm.at[idx_v], rows_v, sem).wait()
    pltpu.sync_copy(rows_v, xd_hbm.at[pl.ds(base, EPW)])


@jax.jit
def _sc_gather(state, src3, dst3):
    return pl.kernel(
        _gather_body,
        out_type=[
            jax.ShapeDtypeStruct((E_PAD, 16), jnp.float32),
            jax.ShapeDtypeStruct((E_PAD, 16), jnp.float32),
        ],
        mesh=_mesh(),
        compiler_params=pltpu.CompilerParams(use_tc_tiling_on_sc=False),
        scratch_types=[
            pltpu.VMEM((EPW,), jnp.int32),
            pltpu.VMEM((EPW, 16), jnp.float32),
            pltpu.SemaphoreType.DMA,
        ],
    )(state, src3, dst3)


def _scatter_body(msgs_hbm, dst_hbm, out_hbm, idx_v, rows_v, zb_v, sem, agg_sh):
    c = lax.axis_index("c")
    s = lax.axis_index("s")
    wid = s * 2 + c

    def _zero(i, carry):
        zb_v[i] = jnp.zeros((16,), jnp.float32)
        return carry

    lax.fori_loop(0, RPT, _zero, 0)
    pltpu.sync_copy(zb_v, agg_sh.at[pl.ds(s * RPT, RPT)])
    plsc.subcore_barrier()
    pltpu.sync_copy(msgs_hbm.at[pl.ds(wid * EPW, EPW)], rows_v)
    pltpu.sync_copy(dst_hbm.at[wid], idx_v)
    pltpu.sync_copy(rows_v, agg_sh.at[idx_v], add=True)
    plsc.subcore_barrier()
    pltpu.sync_copy(agg_sh.at[pl.ds(s * RPT, RPT)],
                    out_hbm.at[c, pl.ds(s * RPT, RPT)])


@jax.jit
def _sc_scatter(msgs, dst3):
    return pl.kernel(
        _scatter_body,
        out_type=jax.ShapeDtypeStruct((2, NP, 16), jnp.float32),
        mesh=_mesh(),
        compiler_params=pltpu.CompilerParams(use_tc_tiling_on_sc=False),
        scratch_types=[
            pltpu.VMEM((EPW,), jnp.int32),
            pltpu.VMEM((EPW, 16), jnp.float32),
            pltpu.VMEM((RPT, 16), jnp.float32),
            pltpu.SemaphoreType.DMA,
            pltpu.VMEM_SHARED((NP, 16), jnp.float32),
        ],
    )(msgs, dst3)


_BE = 4096


def _mlp_body(xs_ref, xd_ref, w1a, w1b, b1, w2, b2, w3, b3, out_ref):
    bf = jnp.bfloat16
    h = jnp.dot(xs_ref[...].astype(bf), w1a[...], preferred_element_type=jnp.float32)
    h = h + jnp.dot(xd_ref[...].astype(bf), w1b[...], preferred_element_type=jnp.float32)
    h = jax.nn.relu(h + b1[...])
    h = jax.nn.relu(jnp.dot(h.astype(bf), w2[...], preferred_element_type=jnp.float32) + b2[...])
    out_ref[...] = jnp.dot(h.astype(bf), w3[...], preferred_element_type=jnp.float32) + b3[...]


@jax.jit
def _tc_mlp(xs, xd, w1a, w1b, b1, w2, b2, w3, b3):
    n_blk = E_PAD // _BE
    full = lambda i: (0, 0)
    return pl.pallas_call(
        _mlp_body,
        grid=(n_blk,),
        in_specs=[
            pl.BlockSpec((_BE, 16), lambda i: (i, 0)),
            pl.BlockSpec((_BE, 16), lambda i: (i, 0)),
            pl.BlockSpec((16, MLP_H), full),
            pl.BlockSpec((16, MLP_H), full),
            pl.BlockSpec((1, MLP_H), full),
            pl.BlockSpec((MLP_H, MLP_H), full),
            pl.BlockSpec((1, MLP_H), full),
            pl.BlockSpec((MLP_H, 16), full),
            pl.BlockSpec((1, 16), full),
        ],
        out_specs=pl.BlockSpec((_BE, 16), lambda i: (i, 0)),
        out_shape=jax.ShapeDtypeStruct((E_PAD, 16), jnp.float32),
    )(xs, xd, w1a, w1b, b1, w2, b2, w3, b3)


def _update_body(agg2_ref, st_ref, ni_ref, wia, wib, whh, bih, bhh, wout, bout,
                 ns_ref, out_ref):
    agg = agg2_ref[0] + agg2_ref[1]
    st = st_ref[...]
    gx = (jnp.dot(agg, wia[...], preferred_element_type=jnp.float32)
          + jnp.dot(ni_ref[...], wib[...], preferred_element_type=jnp.float32)
          + bih[...])
    gh = jnp.dot(st, whh[...], preferred_element_type=jnp.float32) + bhh[...]
    r = jax.nn.sigmoid(gx[:, 0:16] + gh[:, 0:16])
    z = jax.nn.sigmoid(gx[:, 16:32] + gh[:, 16:32])
    n = jnp.tanh(gx[:, 32:48] + r * gh[:, 32:48])
    ns = (1.0 - z) * n + z * st
    ns_ref[...] = ns
    logits = jnp.dot(ns, wout[...], preferred_element_type=jnp.float32) + bout[...]
    col = lax.broadcasted_iota(jnp.int32, logits.shape, 1)
    logits = jnp.where(col < 9, logits, -1e30)
    m = jnp.max(logits, axis=1, keepdims=True)
    e = jnp.exp(logits - m)
    out_ref[...] = e / jnp.sum(e, axis=1, keepdims=True)


@jax.jit
def _tc_update(agg2, state, ni, wia, wib, whh, bih, bhh, wout, bout):
    full = lambda: (0, 0)
    return pl.pallas_call(
        _update_body,
        out_shape=[
            jax.ShapeDtypeStruct((NP, 16), jnp.float32),
            jax.ShapeDtypeStruct((NP, 16), jnp.float32),
        ],
    )(agg2, state, ni, wia, wib, whh, bih, bhh, wout, bout)


def _prep_idx(ids):
    ids = ids.reshape(NW, N_EDGES // NW)
    pad = jnp.full((NW, EPW - N_EDGES // NW), SINK, dtype=jnp.int32)
    return jnp.concatenate([ids, pad], axis=1).reshape(NW, EPW)


def _pad2(a, rows, cols):
    return jnp.zeros((rows, cols), a.dtype).at[: a.shape[0], : a.shape[1]].set(a)


def _gate_pad(wt, in_real):
    out = jnp.zeros((16, 48), wt.dtype)
    for g in range(3):
        out = out.at[:in_real, g * 16:g * 16 + DH].set(wt[:, g * DH:(g + 1) * DH])
    return out


def kernel(node_inputs, src_ids, dst_ids, W1, b1, W2, b2, W3, b3, Wout, bout,
           W_ih, W_hh, b_ih, b_hh):
    f32 = jnp.float32
    src3 = _prep_idx(src_ids)
    dst3 = _prep_idx(dst_ids)
    ni = _pad2(node_inputs.astype(f32), NP, 16)

    bf = jnp.bfloat16
    w1t = W1.T
    w1a = _pad2(w1t[:DH], 16, MLP_H).astype(bf)
    w1b = _pad2(w1t[DH:], 16, MLP_H).astype(bf)
    b1r = b1.reshape(1, MLP_H)
    w2t = W2.T.astype(bf)
    b2r = b2.reshape(1, MLP_H)
    w3t = _pad2(W3.T, MLP_H, 16).astype(bf)
    b3r = _pad2(b3.reshape(1, DE), 1, 16)

    wiht = W_ih.T
    wia = _gate_pad(wiht[:DE], DE)
    wib = _gate_pad(wiht[DE:], DIN)
    whh = _gate_pad(W_hh.T, DH)
    bih = jnp.zeros((1, 48), f32)
    bhh = jnp.zeros((1, 48), f32)
    for g in range(3):
        bih = bih.at[0, g * 16:g * 16 + DH].set(b_ih[g * DH:(g + 1) * DH])
        bhh = bhh.at[0, g * 16:g * 16 + DH].set(b_hh[g * DH:(g + 1) * DH])
    woutt = _pad2(Wout.T, 16, 16)
    boutr = _pad2(bout.reshape(1, DOUT := 9), 1, 16)

    state = jnp.zeros((NP, 16), f32)
    outs = []
    for _ in range(N_ITERS):
        xs, xd = _sc_gather(state, src3, dst3)
        msgs = _tc_mlp(xs, xd, w1a, w1b, b1r, w2t, b2r, w3t, b3r)
        agg2 = _sc_scatter(msgs, dst3)
        state, out_i = _tc_update(agg2, state, ni, wia, wib, whh, bih, bhh,
                                  woutt, boutr)
        outs.append(out_i)
    total = jnp.stack(outs, axis=0)
    return total[:, :N_NODES, :9]


# speedup vs baseline: 2.6625x; 2.6625x over previous
import functools

import jax
import jax.numpy as jnp
from jax import lax
from jax.experimental import pallas as pl
from jax.experimental.pallas import tpu as pltpu
from jax.experimental.pallas import tpu_sc as plsc

N_NODES = 10000
N_EDGES = 160000
N_ITERS = 7
DH = 10
DE = 11
DIN = 9
MLP_H = 96

NP = 10240
NP8 = NP // 8
SINK = N_NODES
NW = 32
EPW = 5120
SPW = EPW // 8
E_PAD = NW * EPW
E8 = E_PAD // 8
RPT = NP // 16
RPT8 = NP8 // 16


@functools.cache
def _mesh():
    return plsc.VectorSubcoreMesh(core_axis_name="c", subcore_axis_name="s")


def _gather_body(state_hbm, src_hbm, dst_hbm, xs_hbm, xd_hbm, idx_v, rows_v, sem):
    wid = lax.axis_index("s") * 2 + lax.axis_index("c")
    pltpu.sync_copy(src_hbm.at[wid], idx_v)
    pltpu.async_copy(state_hbm.at[idx_v], rows_v, sem).wait()
    pltpu.sync_copy(rows_v, xs_hbm.at[wid])
    pltpu.sync_copy(dst_hbm.at[wid], idx_v)
    pltpu.async_copy(state_hbm.at[idx_v], rows_v, sem).wait()
    pltpu.sync_copy(rows_v, xd_hbm.at[wid])


@jax.jit
def _sc_gather(state, src3, dst3):
    return pl.kernel(
        _gather_body,
        out_type=[
            jax.ShapeDtypeStruct((NW, EPW, 16), jnp.float32),
            jax.ShapeDtypeStruct((NW, EPW, 16), jnp.float32),
        ],
        mesh=_mesh(),
        compiler_params=pltpu.CompilerParams(use_tc_tiling_on_sc=False),
        scratch_types=[
            pltpu.VMEM((EPW,), jnp.int32),
            pltpu.VMEM((EPW, 16), jnp.float32),
            pltpu.SemaphoreType.DMA,
        ],
    )(state, src3, dst3)


def _scatter_body(msgs_hbm, dst_hbm, out_hbm, idx_v, rows_v, zb_v, sem, agg_sh):
    c = lax.axis_index("c")
    s = lax.axis_index("s")
    wid = s * 2 + c

    def _zero(i, carry):
        zb_v[i] = jnp.zeros((16,), jnp.float32)
        return carry

    lax.fori_loop(0, RPT, _zero, 0)
    pltpu.sync_copy(zb_v, agg_sh.at[pl.ds(s * RPT, RPT)])
    plsc.subcore_barrier()
    pltpu.sync_copy(msgs_hbm.at[wid], rows_v)
    pltpu.sync_copy(dst_hbm.at[wid], idx_v)
    pltpu.sync_copy(rows_v, agg_sh.at[idx_v], add=True)
    plsc.subcore_barrier()
    pltpu.sync_copy(agg_sh.at[pl.ds(s * RPT, RPT)],
                    out_hbm.at[c, pl.ds(s * RPT, RPT)])


@jax.jit
def _sc_scatter(msgs3, dst3):
    return pl.kernel(
        _scatter_body,
        out_type=jax.ShapeDtypeStruct((2, NP, 16), jnp.float32),
        mesh=_mesh(),
        compiler_params=pltpu.CompilerParams(use_tc_tiling_on_sc=False),
        scratch_types=[
            pltpu.VMEM((EPW,), jnp.int32),
            pltpu.VMEM((EPW, 16), jnp.float32),
            pltpu.VMEM((RPT, 16), jnp.float32),
            pltpu.SemaphoreType.DMA,
            pltpu.VMEM_SHARED((NP, 16), jnp.float32),
        ],
    )(msgs3, dst3)


_BR = 512


def _unpack8(x8):
    t = x8.T
    return jnp.concatenate([t[16 * j:16 * j + 16, :] for j in range(8)], axis=1)


def _pack8(m, rows):
    t = jnp.concatenate([m[:, rows * j:rows * (j + 1)] for j in range(8)], axis=0)
    return t.T


def _mlp_body(xs_ref, xd_ref, w1a, w1b, b1, w2, b2, w3, b3, out_ref):
    bf = jnp.bfloat16
    f32 = jnp.float32
    xs = _unpack8(xs_ref[...]).astype(bf)
    xd = _unpack8(xd_ref[...]).astype(bf)
    h = jnp.dot(w1a[...], xs, preferred_element_type=f32)
    h = h + jnp.dot(w1b[...], xd, preferred_element_type=f32)
    h = jax.nn.relu(h + b1[...])
    h = jax.nn.relu(jnp.dot(w2[...], h.astype(bf), preferred_element_type=f32)
                    + b2[...])
    m = jnp.dot(w3[...], h.astype(bf), preferred_element_type=f32) + b3[...]
    out_ref[...] = _pack8(m, _BR)


@jax.jit
def _tc_mlp(xs, xd, w1a, w1b, b1, w2, b2, w3, b3):
    n_blk = E8 // _BR
    full = lambda i: (0, 0)
    return pl.pallas_call(
        _mlp_body,
        grid=(n_blk,),
        in_specs=[
            pl.BlockSpec((_BR, 128), lambda i: (i, 0)),
            pl.BlockSpec((_BR, 128), lambda i: (i, 0)),
            pl.BlockSpec((MLP_H, 16), full),
            pl.BlockSpec((MLP_H, 16), full),
            pl.BlockSpec((MLP_H, 1), full),
            pl.BlockSpec((MLP_H, MLP_H), full),
            pl.BlockSpec((MLP_H, 1), full),
            pl.BlockSpec((16, MLP_H), full),
            pl.BlockSpec((16, 1), full),
        ],
        out_specs=pl.BlockSpec((_BR, 128), lambda i: (i, 0)),
        out_shape=jax.ShapeDtypeStruct((E8, 128), jnp.float32),
    )(xs, xd, w1a, w1b, b1, w2, b2, w3, b3)


def _update_body(agg2_ref, st_ref, ni_ref, wia, wib, whh, bih, bhh, wout, bout,
                 ns_ref, out_ref):
    f32 = jnp.float32
    agg = _unpack8(agg2_ref[0] + agg2_ref[1])
    st = _unpack8(st_ref[...])
    gx = (jnp.dot(wia[...], agg, preferred_element_type=f32)
          + jnp.dot(wib[...], ni_ref[...], preferred_element_type=f32)
          + bih[...])
    gh = jnp.dot(whh[...], st, preferred_element_type=f32) + bhh[...]
    r = jax.nn.sigmoid(gx[0:16] + gh[0:16])
    z = jax.nn.sigmoid(gx[16:32] + gh[16:32])
    n = jnp.tanh(gx[32:48] + r * gh[32:48])
    ns = (1.0 - z) * n + z * st
    ns_ref[...] = _pack8(ns, NP8)
    logits = jnp.dot(wout[...], ns, preferred_element_type=f32) + bout[...]
    row = lax.broadcasted_iota(jnp.int32, logits.shape, 0)
    logits = jnp.where(row < 9, logits, -1e30)
    m = jnp.max(logits, axis=0, keepdims=True)
    e = jnp.exp(logits - m)
    out_ref[...] = e / jnp.sum(e, axis=0, keepdims=True)


@jax.jit
def _tc_update(agg2, state8, niT, wia, wib, whh, bih, bhh, wout, bout):
    return pl.pallas_call(
        _update_body,
        out_shape=[
            jax.ShapeDtypeStruct((NP8, 128), jnp.float32),
            jax.ShapeDtypeStruct((16, NP), jnp.float32),
        ],
    )(agg2, state8, niT, wia, wib, whh, bih, bhh, wout, bout)


def _prep_idx(ids):
    ids = ids.reshape(NW, N_EDGES // NW)
    pad = jnp.full((NW, EPW - N_EDGES // NW), SINK, dtype=jnp.int32)
    return jnp.concatenate([ids, pad], axis=1)


def _pad2(a, rows, cols):
    return jnp.zeros((rows, cols), a.dtype).at[: a.shape[0], : a.shape[1]].set(a)


def _gate_pad_rows(w, in_real):
    out = jnp.zeros((48, 16), w.dtype)
    for g in range(3):
        out = out.at[g * 16:g * 16 + DH, :in_real].set(w[g * DH:(g + 1) * DH])
    return out


def kernel(node_inputs, src_ids, dst_ids, W1, b1, W2, b2, W3, b3, Wout, bout,
           W_ih, W_hh, b_ih, b_hh):
    f32 = jnp.float32
    bf = jnp.bfloat16
    src3 = _prep_idx(src_ids)
    dst3 = _prep_idx(dst_ids)

    ni = _pad2(node_inputs.astype(f32), NP, 16)
    niT = ni.reshape(NP8, 8, 16).transpose(2, 1, 0).reshape(16, NP)

    w1a = _pad2(W1[:, :DH], MLP_H, 16).astype(bf)
    w1b = _pad2(W1[:, DH:], MLP_H, 16).astype(bf)
    b1c = b1.reshape(MLP_H, 1)
    w2m = W2.astype(bf)
    b2c = b2.reshape(MLP_H, 1)
    w3m = _pad2(W3, 16, MLP_H).astype(bf)
    b3c = _pad2(b3.reshape(DE, 1), 16, 1)

    wia = _gate_pad_rows(W_ih[:, :DE], DE)
    wib = _gate_pad_rows(W_ih[:, DE:], DIN)
    whh = _gate_pad_rows(W_hh, DH)
    bih = jnp.zeros((48, 1), f32)
    bhh = jnp.zeros((48, 1), f32)
    for g in range(3):
        bih = bih.at[g * 16:g * 16 + DH, 0].set(b_ih[g * DH:(g + 1) * DH])
        bhh = bhh.at[g * 16:g * 16 + DH, 0].set(b_hh[g * DH:(g + 1) * DH])
    woutp = _pad2(Wout, 16, 16)
    boutc = _pad2(bout.reshape(9, 1), 16, 1)

    state8 = jnp.zeros((NP8, 128), f32)
    outs = []
    for _ in range(N_ITERS):
        xs, xd = _sc_gather(state8.reshape(NP, 16), src3, dst3)
        msgs = _tc_mlp(xs.reshape(E8, 128), xd.reshape(E8, 128),
                       w1a, w1b, b1c, w2m, b2c, w3m, b3c)
        agg2 = _sc_scatter(msgs.reshape(NW, EPW, 16), dst3)
        state8, out_i = _tc_update(agg2.reshape(2, NP8, 128), state8, niT,
                                   wia, wib, whh, bih, bhh, woutp, boutc)
        outs.append(out_i)
    total = jnp.stack(outs, axis=0)
    total = total.reshape(N_ITERS, 16, 8, NP8).transpose(0, 3, 2, 1)
    return total.reshape(N_ITERS, NP, 16)[:, :N_NODES, :9]


# speedup vs baseline: 2.6917x; 1.0110x over previous
import functools

import jax
import jax.numpy as jnp
from jax import lax
from jax.experimental import pallas as pl
from jax.experimental.pallas import tpu as pltpu
from jax.experimental.pallas import tpu_sc as plsc

N_NODES = 10000
N_EDGES = 160000
N_ITERS = 7
DH = 10
DE = 11
DIN = 9
MLP_H = 96

NP = 10240
NP8 = NP // 8
SINK = N_NODES
NW = 32
EPW = 5120
SPW = EPW // 8
E_PAD = NW * EPW
E8 = E_PAD // 8
RPT = NP // 16
RPT8 = NP8 // 16


@functools.cache
def _mesh():
    return plsc.VectorSubcoreMesh(core_axis_name="c", subcore_axis_name="s")


_NB = 4
_JR = 1280


def _gather_body(state_hbm, src_hbm, dst_hbm, xs_hbm, xd_hbm, idx_v,
                 b0, b1, b2, b3, gs0, gs1, gs2, gs3, cs0, cs1, cs2, cs3):
    wid = lax.axis_index("s") * 2 + lax.axis_index("c")
    bufs = (b0, b1, b2, b3)
    gsems = (gs0, gs1, gs2, gs3)
    csems = (cs0, cs1, cs2, cs3)
    pltpu.sync_copy(src_hbm.at[wid], idx_v.at[pl.ds(0, EPW)])
    pltpu.sync_copy(dst_hbm.at[wid], idx_v.at[pl.ds(EPW, EPW)])

    def _out_slab(j):
        tgt = xs_hbm if j < 4 else xd_hbm
        return tgt.at[wid, pl.ds((j % 4) * _JR, _JR)]

    gathers = [None] * 8
    copies = [None] * 8
    for j in range(8):
        if j >= _NB:
            copies[j - _NB].wait()
        gathers[j] = pltpu.async_copy(
            state_hbm.at[idx_v.at[pl.ds(j * _JR, _JR)]], bufs[j % _NB],
            gsems[j % _NB])
        if j >= 1:
            k = j - 1
            gathers[k].wait()
            copies[k] = pltpu.async_copy(bufs[k % _NB], _out_slab(k),
                                         csems[k % _NB])
    gathers[7].wait()
    copies[7] = pltpu.async_copy(bufs[3], _out_slab(7), csems[3])
    for k in range(4, 8):
        copies[k].wait()


@jax.jit
def _sc_gather(state, src3, dst3):
    return pl.kernel(
        _gather_body,
        out_type=[
            jax.ShapeDtypeStruct((NW, EPW, 16), jnp.float32),
            jax.ShapeDtypeStruct((NW, EPW, 16), jnp.float32),
        ],
        mesh=_mesh(),
        compiler_params=pltpu.CompilerParams(use_tc_tiling_on_sc=False),
        scratch_types=[
            pltpu.VMEM((2 * EPW,), jnp.int32),
        ] + [pltpu.VMEM((_JR, 16), jnp.float32)] * 4
          + [pltpu.SemaphoreType.DMA] * 8,
    )(state, src3, dst3)


def _scatter_body(msgs_hbm, dst_hbm, out_hbm, idx_v, rows_v, zb_v, sem, agg_sh):
    c = lax.axis_index("c")
    s = lax.axis_index("s")
    wid = s * 2 + c

    def _zero(i, carry):
        zb_v[i] = jnp.zeros((16,), jnp.float32)
        return carry

    lax.fori_loop(0, RPT, _zero, 0)
    pltpu.sync_copy(zb_v, agg_sh.at[pl.ds(s * RPT, RPT)])
    plsc.subcore_barrier()
    pltpu.sync_copy(msgs_hbm.at[wid], rows_v)
    pltpu.sync_copy(dst_hbm.at[wid], idx_v)
    pltpu.sync_copy(rows_v, agg_sh.at[idx_v], add=True)
    plsc.subcore_barrier()
    pltpu.sync_copy(agg_sh.at[pl.ds(s * RPT, RPT)],
                    out_hbm.at[c, pl.ds(s * RPT, RPT)])


@jax.jit
def _sc_scatter(msgs3, dst3):
    return pl.kernel(
        _scatter_body,
        out_type=jax.ShapeDtypeStruct((2, NP, 16), jnp.float32),
        mesh=_mesh(),
        compiler_params=pltpu.CompilerParams(use_tc_tiling_on_sc=False),
        scratch_types=[
            pltpu.VMEM((EPW,), jnp.int32),
            pltpu.VMEM((EPW, 16), jnp.float32),
            pltpu.VMEM((RPT, 16), jnp.float32),
            pltpu.SemaphoreType.DMA,
            pltpu.VMEM_SHARED((NP, 16), jnp.float32),
        ],
    )(msgs3, dst3)


_BR = 512


def _unpack8(x8):
    t = x8.T
    return jnp.concatenate([t[16 * j:16 * j + 16, :] for j in range(8)], axis=1)


def _pack8(m, rows):
    t = jnp.concatenate([m[:, rows * j:rows * (j + 1)] for j in range(8)], axis=0)
    return t.T


def _mlp_body(xs_ref, xd_ref, w1a, w1b, b1, w2, b2, w3, b3, out_ref):
    bf = jnp.bfloat16
    f32 = jnp.float32
    xs = _unpack8(xs_ref[...]).astype(bf)
    xd = _unpack8(xd_ref[...]).astype(bf)
    h = jnp.dot(w1a[...], xs, preferred_element_type=f32)
    h = h + jnp.dot(w1b[...], xd, preferred_element_type=f32)
    h = jax.nn.relu(h + b1[...])
    h = jax.nn.relu(jnp.dot(w2[...], h.astype(bf), preferred_element_type=f32)
                    + b2[...])
    m = jnp.dot(w3[...], h.astype(bf), preferred_element_type=f32) + b3[...]
    out_ref[...] = _pack8(m, _BR)


@jax.jit
def _tc_mlp(xs, xd, w1a, w1b, b1, w2, b2, w3, b3):
    n_blk = E8 // _BR
    full = lambda i: (0, 0)
    return pl.pallas_call(
        _mlp_body,
        grid=(n_blk,),
        in_specs=[
            pl.BlockSpec((_BR, 128), lambda i: (i, 0)),
            pl.BlockSpec((_BR, 128), lambda i: (i, 0)),
            pl.BlockSpec((MLP_H, 16), full),
            pl.BlockSpec((MLP_H, 16), full),
            pl.BlockSpec((MLP_H, 1), full),
            pl.BlockSpec((MLP_H, MLP_H), full),
            pl.BlockSpec((MLP_H, 1), full),
            pl.BlockSpec((16, MLP_H), full),
            pl.BlockSpec((16, 1), full),
        ],
        out_specs=pl.BlockSpec((_BR, 128), lambda i: (i, 0)),
        out_shape=jax.ShapeDtypeStruct((E8, 128), jnp.float32),
    )(xs, xd, w1a, w1b, b1, w2, b2, w3, b3)


def _update_body(agg2_ref, st_ref, ni_ref, wia, wib, whh, bih, bhh, wout, bout,
                 ns_ref, out_ref):
    f32 = jnp.float32
    agg = _unpack8(agg2_ref[0] + agg2_ref[1])
    st = _unpack8(st_ref[...])
    gx = (jnp.dot(wia[...], agg, preferred_element_type=f32)
          + jnp.dot(wib[...], ni_ref[...], preferred_element_type=f32)
          + bih[...])
    gh = jnp.dot(whh[...], st, preferred_element_type=f32) + bhh[...]
    r = jax.nn.sigmoid(gx[0:16] + gh[0:16])
    z = jax.nn.sigmoid(gx[16:32] + gh[16:32])
    n = jnp.tanh(gx[32:48] + r * gh[32:48])
    ns = (1.0 - z) * n + z * st
    ns_ref[...] = _pack8(ns, NP8)
    logits = jnp.dot(wout[...], ns, preferred_element_type=f32) + bout[...]
    row = lax.broadcasted_iota(jnp.int32, logits.shape, 0)
    logits = jnp.where(row < 9, logits, -1e30)
    m = jnp.max(logits, axis=0, keepdims=True)
    e = jnp.exp(logits - m)
    out_ref[...] = e / jnp.sum(e, axis=0, keepdims=True)


@jax.jit
def _tc_update(agg2, state8, niT, wia, wib, whh, bih, bhh, wout, bout):
    return pl.pallas_call(
        _update_body,
        out_shape=[
            jax.ShapeDtypeStruct((NP8, 128), jnp.float32),
            jax.ShapeDtypeStruct((16, NP), jnp.float32),
        ],
    )(agg2, state8, niT, wia, wib, whh, bih, bhh, wout, bout)


def _prep_idx(ids):
    ids = ids.reshape(NW, N_EDGES // NW)
    pad = jnp.full((NW, EPW - N_EDGES // NW), SINK, dtype=jnp.int32)
    return jnp.concatenate([ids, pad], axis=1)


def _pad2(a, rows, cols):
    return jnp.zeros((rows, cols), a.dtype).at[: a.shape[0], : a.shape[1]].set(a)


def _gate_pad_rows(w, in_real):
    out = jnp.zeros((48, 16), w.dtype)
    for g in range(3):
        out = out.at[g * 16:g * 16 + DH, :in_real].set(w[g * DH:(g + 1) * DH])
    return out


def kernel(node_inputs, src_ids, dst_ids, W1, b1, W2, b2, W3, b3, Wout, bout,
           W_ih, W_hh, b_ih, b_hh):
    f32 = jnp.float32
    bf = jnp.bfloat16
    src3 = _prep_idx(src_ids)
    dst3 = _prep_idx(dst_ids)

    ni = _pad2(node_inputs.astype(f32), NP, 16)
    niT = ni.reshape(NP8, 8, 16).transpose(2, 1, 0).reshape(16, NP)

    w1a = _pad2(W1[:, :DH], MLP_H, 16).astype(bf)
    w1b = _pad2(W1[:, DH:], MLP_H, 16).astype(bf)
    b1c = b1.reshape(MLP_H, 1)
    w2m = W2.astype(bf)
    b2c = b2.reshape(MLP_H, 1)
    w3m = _pad2(W3, 16, MLP_H).astype(bf)
    b3c = _pad2(b3.reshape(DE, 1), 16, 1)

    wia = _gate_pad_rows(W_ih[:, :DE], DE)
    wib = _gate_pad_rows(W_ih[:, DE:], DIN)
    whh = _gate_pad_rows(W_hh, DH)
    bih = jnp.zeros((48, 1), f32)
    bhh = jnp.zeros((48, 1), f32)
    for g in range(3):
        bih = bih.at[g * 16:g * 16 + DH, 0].set(b_ih[g * DH:(g + 1) * DH])
        bhh = bhh.at[g * 16:g * 16 + DH, 0].set(b_hh[g * DH:(g + 1) * DH])
    woutp = _pad2(Wout, 16, 16)
    boutc = _pad2(bout.reshape(9, 1), 16, 1)

    state8 = jnp.zeros((NP8, 128), f32)
    outs = []
    for _ in range(N_ITERS):
        xs, xd = _sc_gather(state8.reshape(NP, 16), src3, dst3)
        msgs = _tc_mlp(xs.reshape(E8, 128), xd.reshape(E8, 128),
                       w1a, w1b, b1c, w2m, b2c, w3m, b3c)
        agg2 = _sc_scatter(msgs.reshape(NW, EPW, 16), dst3)
        state8, out_i = _tc_update(agg2.reshape(2, NP8, 128), state8, niT,
                                   wia, wib, whh, bih, bhh, woutp, boutc)
        outs.append(out_i)
    total = jnp.stack(outs, axis=0)
    total = total.reshape(N_ITERS, 16, 8, NP8).transpose(0, 3, 2, 1)
    return total.reshape(N_ITERS, NP, 16)[:, :N_NODES, :9]


# speedup vs baseline: 3.8399x; 1.4266x over previous
import functools

import jax
import jax.numpy as jnp
from jax import lax
from jax.experimental import pallas as pl
from jax.experimental.pallas import tpu as pltpu
from jax.experimental.pallas import tpu_sc as plsc

N_NODES = 10000
N_EDGES = 160000
N_ITERS = 7
DH = 10
DE = 11
DIN = 9
MLP_H = 96

NP = 10240
NP8 = NP // 8
SINK = N_NODES
NW = 32
EPW = 5120
SPW = EPW // 8
E_PAD = NW * EPW
E8 = E_PAD // 8
RPT = NP // 16
RPT8 = NP8 // 16


@functools.cache
def _mesh():
    return plsc.VectorSubcoreMesh(core_axis_name="c", subcore_axis_name="s")


_NB = 4
_JR = 1280


def _gather_body(state_hbm, src_hbm, dst_hbm, xs_hbm, xd_hbm, idx_v,
                 b0, b1, b2, b3, gs0, gs1, gs2, gs3, cs0, cs1, cs2, cs3,
                 state_sh):
    c = lax.axis_index("c")
    s = lax.axis_index("s")
    wid = s * 2 + c
    bufs = (b0, b1, b2, b3)
    gsems = (gs0, gs1, gs2, gs3)
    csems = (cs0, cs1, cs2, cs3)
    pltpu.sync_copy(state_hbm.at[pl.ds(s * RPT, RPT)],
                    state_sh.at[pl.ds(s * RPT, RPT)])
    pltpu.sync_copy(src_hbm.at[wid], idx_v.at[pl.ds(0, EPW)])
    pltpu.sync_copy(dst_hbm.at[wid], idx_v.at[pl.ds(EPW, EPW)])
    plsc.subcore_barrier()

    def _out_slab(j):
        tgt = xs_hbm if j < 4 else xd_hbm
        return tgt.at[wid, pl.ds((j % 4) * _JR, _JR)]

    gathers = [None] * 8
    copies = [None] * 8
    for j in range(8):
        if j >= _NB:
            copies[j - _NB].wait()
        gathers[j] = pltpu.async_copy(
            state_sh.at[idx_v.at[pl.ds(j * _JR, _JR)]], bufs[j % _NB],
            gsems[j % _NB])
        if j >= 1:
            k = j - 1
            gathers[k].wait()
            copies[k] = pltpu.async_copy(bufs[k % _NB], _out_slab(k),
                                         csems[k % _NB])
    gathers[7].wait()
    copies[7] = pltpu.async_copy(bufs[3], _out_slab(7), csems[3])
    for k in range(4, 8):
        copies[k].wait()


@jax.jit
def _sc_gather(state, src3, dst3):
    return pl.kernel(
        _gather_body,
        out_type=[
            jax.ShapeDtypeStruct((NW, EPW, 16), jnp.float32),
            jax.ShapeDtypeStruct((NW, EPW, 16), jnp.float32),
        ],
        mesh=_mesh(),
        compiler_params=pltpu.CompilerParams(use_tc_tiling_on_sc=False),
        scratch_types=[
            pltpu.VMEM((2 * EPW,), jnp.int32),
        ] + [pltpu.VMEM((_JR, 16), jnp.float32)] * 4
          + [pltpu.SemaphoreType.DMA] * 8
          + [pltpu.VMEM_SHARED((NP, 16), jnp.float32)],
    )(state, src3, dst3)


def _scatter_body(msgs_hbm, dst_hbm, out_hbm, idx_v, rows_v, zb_v, sem, agg_sh):
    c = lax.axis_index("c")
    s = lax.axis_index("s")
    wid = s * 2 + c

    def _zero(i, carry):
        zb_v[i] = jnp.zeros((16,), jnp.float32)
        return carry

    lax.fori_loop(0, RPT, _zero, 0)
    pltpu.sync_copy(zb_v, agg_sh.at[pl.ds(s * RPT, RPT)])
    plsc.subcore_barrier()
    pltpu.sync_copy(msgs_hbm.at[wid], rows_v)
    pltpu.sync_copy(dst_hbm.at[wid], idx_v)
    pltpu.sync_copy(rows_v, agg_sh.at[idx_v], add=True)
    plsc.subcore_barrier()
    pltpu.sync_copy(agg_sh.at[pl.ds(s * RPT, RPT)],
                    out_hbm.at[c, pl.ds(s * RPT, RPT)])


@jax.jit
def _sc_scatter(msgs3, dst3):
    return pl.kernel(
        _scatter_body,
        out_type=jax.ShapeDtypeStruct((2, NP, 16), jnp.float32),
        mesh=_mesh(),
        compiler_params=pltpu.CompilerParams(use_tc_tiling_on_sc=False),
        scratch_types=[
            pltpu.VMEM((EPW,), jnp.int32),
            pltpu.VMEM((EPW, 16), jnp.float32),
            pltpu.VMEM((RPT, 16), jnp.float32),
            pltpu.SemaphoreType.DMA,
            pltpu.VMEM_SHARED((NP, 16), jnp.float32),
        ],
    )(msgs3, dst3)


_BR = 512


def _unpack8(x8):
    t = x8.T
    return jnp.concatenate([t[16 * j:16 * j + 16, :] for j in range(8)], axis=1)


def _pack8(m, rows):
    t = jnp.concatenate([m[:, rows * j:rows * (j + 1)] for j in range(8)], axis=0)
    return t.T


def _mlp_body(xs_ref, xd_ref, w1a, w1b, b1, w2, b2, w3, b3, out_ref):
    bf = jnp.bfloat16
    f32 = jnp.float32
    xs = _unpack8(xs_ref[...]).astype(bf)
    xd = _unpack8(xd_ref[...]).astype(bf)
    h = jnp.dot(w1a[...], xs, preferred_element_type=f32)
    h = h + jnp.dot(w1b[...], xd, preferred_element_type=f32)
    h = jax.nn.relu(h + b1[...])
    h = jax.nn.relu(jnp.dot(w2[...], h.astype(bf), preferred_element_type=f32)
                    + b2[...])
    m = jnp.dot(w3[...], h.astype(bf), preferred_element_type=f32) + b3[...]
    out_ref[...] = _pack8(m, _BR)


@jax.jit
def _tc_mlp(xs, xd, w1a, w1b, b1, w2, b2, w3, b3):
    n_blk = E8 // _BR
    full = lambda i: (0, 0)
    return pl.pallas_call(
        _mlp_body,
        grid=(n_blk,),
        in_specs=[
            pl.BlockSpec((_BR, 128), lambda i: (i, 0)),
            pl.BlockSpec((_BR, 128), lambda i: (i, 0)),
            pl.BlockSpec((MLP_H, 16), full),
            pl.BlockSpec((MLP_H, 16), full),
            pl.BlockSpec((MLP_H, 1), full),
            pl.BlockSpec((MLP_H, MLP_H), full),
            pl.BlockSpec((MLP_H, 1), full),
            pl.BlockSpec((16, MLP_H), full),
            pl.BlockSpec((16, 1), full),
        ],
        out_specs=pl.BlockSpec((_BR, 128), lambda i: (i, 0)),
        out_shape=jax.ShapeDtypeStruct((E8, 128), jnp.float32),
    )(xs, xd, w1a, w1b, b1, w2, b2, w3, b3)


def _update_body(agg2_ref, st_ref, ni_ref, wia, wib, whh, bih, bhh, wout, bout,
                 ns_ref, out_ref):
    f32 = jnp.float32
    agg = _unpack8(agg2_ref[0] + agg2_ref[1])
    st = _unpack8(st_ref[...])
    gx = (jnp.dot(wia[...], agg, preferred_element_type=f32)
          + jnp.dot(wib[...], ni_ref[...], preferred_element_type=f32)
          + bih[...])
    gh = jnp.dot(whh[...], st, preferred_element_type=f32) + bhh[...]
    r = jax.nn.sigmoid(gx[0:16] + gh[0:16])
    z = jax.nn.sigmoid(gx[16:32] + gh[16:32])
    n = jnp.tanh(gx[32:48] + r * gh[32:48])
    ns = (1.0 - z) * n + z * st
    ns_ref[...] = _pack8(ns, NP8)
    logits = jnp.dot(wout[...], ns, preferred_element_type=f32) + bout[...]
    row = lax.broadcasted_iota(jnp.int32, logits.shape, 0)
    logits = jnp.where(row < 9, logits, -1e30)
    m = jnp.max(logits, axis=0, keepdims=True)
    e = jnp.exp(logits - m)
    out_ref[...] = e / jnp.sum(e, axis=0, keepdims=True)


@jax.jit
def _tc_update(agg2, state8, niT, wia, wib, whh, bih, bhh, wout, bout):
    return pl.pallas_call(
        _update_body,
        out_shape=[
            jax.ShapeDtypeStruct((NP8, 128), jnp.float32),
            jax.ShapeDtypeStruct((16, NP), jnp.float32),
        ],
    )(agg2, state8, niT, wia, wib, whh, bih, bhh, wout, bout)


def _prep_idx(ids):
    ids = ids.reshape(NW, N_EDGES // NW)
    pad = jnp.full((NW, EPW - N_EDGES // NW), SINK, dtype=jnp.int32)
    return jnp.concatenate([ids, pad], axis=1)


def _pad2(a, rows, cols):
    return jnp.zeros((rows, cols), a.dtype).at[: a.shape[0], : a.shape[1]].set(a)


def _gate_pad_rows(w, in_real):
    out = jnp.zeros((48, 16), w.dtype)
    for g in range(3):
        out = out.at[g * 16:g * 16 + DH, :in_real].set(w[g * DH:(g + 1) * DH])
    return out


def kernel(node_inputs, src_ids, dst_ids, W1, b1, W2, b2, W3, b3, Wout, bout,
           W_ih, W_hh, b_ih, b_hh):
    f32 = jnp.float32
    bf = jnp.bfloat16
    src3 = _prep_idx(src_ids)
    dst3 = _prep_idx(dst_ids)

    ni = _pad2(node_inputs.astype(f32), NP, 16)
    niT = ni.reshape(NP8, 8, 16).transpose(2, 1, 0).reshape(16, NP)

    w1a = _pad2(W1[:, :DH], MLP_H, 16).astype(bf)
    w1b = _pad2(W1[:, DH:], MLP_H, 16).astype(bf)
    b1c = b1.reshape(MLP_H, 1)
    w2m = W2.astype(bf)
    b2c = b2.reshape(MLP_H, 1)
    w3m = _pad2(W3, 16, MLP_H).astype(bf)
    b3c = _pad2(b3.reshape(DE, 1), 16, 1)

    wia = _gate_pad_rows(W_ih[:, :DE], DE)
    wib = _gate_pad_rows(W_ih[:, DE:], DIN)
    whh = _gate_pad_rows(W_hh, DH)
    bih = jnp.zeros((48, 1), f32)
    bhh = jnp.zeros((48, 1), f32)
    for g in range(3):
        bih = bih.at[g * 16:g * 16 + DH, 0].set(b_ih[g * DH:(g + 1) * DH])
        bhh = bhh.at[g * 16:g * 16 + DH, 0].set(b_hh[g * DH:(g + 1) * DH])
    woutp = _pad2(Wout, 16, 16)
    boutc = _pad2(bout.reshape(9, 1), 16, 1)

    state8 = jnp.zeros((NP8, 128), f32)
    outs = []
    for _ in range(N_ITERS):
        xs, xd = _sc_gather(state8.reshape(NP, 16), src3, dst3)
        msgs = _tc_mlp(xs.reshape(E8, 128), xd.reshape(E8, 128),
                       w1a, w1b, b1c, w2m, b2c, w3m, b3c)
        agg2 = _sc_scatter(msgs.reshape(NW, EPW, 16), dst3)
        state8, out_i = _tc_update(agg2.reshape(2, NP8, 128), state8, niT,
                                   wia, wib, whh, bih, bhh, woutp, boutc)
        outs.append(out_i)
    total = jnp.stack(outs, axis=0)
    total = total.reshape(N_ITERS, 16, 8, NP8).transpose(0, 3, 2, 1)
    return total.reshape(N_ITERS, NP, 16)[:, :N_NODES, :9]


# speedup vs baseline: 4.6458x; 1.2099x over previous
import functools

import jax
import jax.numpy as jnp
from jax import lax
from jax.experimental import pallas as pl
from jax.experimental.pallas import tpu as pltpu
from jax.experimental.pallas import tpu_sc as plsc

N_NODES = 10000
N_EDGES = 160000
N_ITERS = 7
DH = 10
DE = 11
DIN = 9
MLP_H = 96

NP = 10240
NP8 = NP // 8
SINK = N_NODES
NW = 32
EPW = 5120
SPW = EPW // 8
E_PAD = NW * EPW
E8 = E_PAD // 8
RPT = NP // 16
RPT8 = NP8 // 16


@functools.cache
def _mesh():
    return plsc.VectorSubcoreMesh(core_axis_name="c", subcore_axis_name="s")


_NB = 4
_JR = 1280


def _gather_body(state_hbm, src_hbm, dst_hbm, xs_hbm, xd_hbm, idx_v,
                 b0, b1, b2, b3, gs0, gs1, gs2, gs3, cs0, cs1, cs2, cs3,
                 state_sh):
    c = lax.axis_index("c")
    s = lax.axis_index("s")
    wid = s * 2 + c
    bufs = (b0, b1, b2, b3)
    gsems = (gs0, gs1, gs2, gs3)
    csems = (cs0, cs1, cs2, cs3)
    pltpu.sync_copy(state_hbm.at[pl.ds(s * RPT, RPT)],
                    state_sh.at[pl.ds(s * RPT, RPT)])
    pltpu.sync_copy(src_hbm.at[wid], idx_v.at[pl.ds(0, EPW)])
    pltpu.sync_copy(dst_hbm.at[wid], idx_v.at[pl.ds(EPW, EPW)])
    plsc.subcore_barrier()

    def _out_slab(j):
        tgt = xs_hbm if j < 4 else xd_hbm
        return tgt.at[wid, pl.ds((j % 4) * _JR, _JR)]

    gathers = [None] * 8
    copies = [None] * 8
    for j in range(8):
        if j >= _NB:
            copies[j - _NB].wait()
        gathers[j] = pltpu.async_copy(
            state_sh.at[idx_v.at[pl.ds(j * _JR, _JR)]], bufs[j % _NB],
            gsems[j % _NB])
        if j >= 1:
            k = j - 1
            gathers[k].wait()
            copies[k] = pltpu.async_copy(bufs[k % _NB], _out_slab(k),
                                         csems[k % _NB])
    gathers[7].wait()
    copies[7] = pltpu.async_copy(bufs[3], _out_slab(7), csems[3])
    for k in range(4, 8):
        copies[k].wait()


@jax.jit
def _sc_gather(state, src3, dst3):
    return pl.kernel(
        _gather_body,
        out_type=[
            jax.ShapeDtypeStruct((NW, EPW, 16), jnp.float32),
            jax.ShapeDtypeStruct((NW, EPW, 16), jnp.float32),
        ],
        mesh=_mesh(),
        compiler_params=pltpu.CompilerParams(use_tc_tiling_on_sc=False),
        scratch_types=[
            pltpu.VMEM((2 * EPW,), jnp.int32),
        ] + [pltpu.VMEM((_JR, 16), jnp.float32)] * 4
          + [pltpu.SemaphoreType.DMA] * 8
          + [pltpu.VMEM_SHARED((NP, 16), jnp.float32)],
    )(state, src3, dst3)


def _scatter_body(msgs_hbm, dst_hbm, out_hbm, idx_v, rows_v, zb_v, sem, agg_sh):
    c = lax.axis_index("c")
    s = lax.axis_index("s")
    wid = s * 2 + c

    def _zero(i, carry):
        zb_v[i] = jnp.zeros((16,), jnp.float32)
        return carry

    lax.fori_loop(0, RPT, _zero, 0)
    pltpu.sync_copy(zb_v, agg_sh.at[pl.ds(s * RPT, RPT)])
    plsc.subcore_barrier()
    pltpu.sync_copy(msgs_hbm.at[wid], rows_v)
    pltpu.sync_copy(dst_hbm.at[wid], idx_v)
    pltpu.sync_copy(rows_v, agg_sh.at[idx_v], add=True)
    plsc.subcore_barrier()
    pltpu.sync_copy(agg_sh.at[pl.ds(s * RPT, RPT)],
                    out_hbm.at[c, pl.ds(s * RPT, RPT)])


def _indeg_body(dst_hbm, out_hbm, idx_v, ones_v, zb_v, agg_sh):
    c = lax.axis_index("c")
    s = lax.axis_index("s")
    wid = s * 2 + c

    def _zero(i, carry):
        zb_v[i] = jnp.zeros((16,), jnp.float32)
        return carry

    def _one(i, carry):
        ones_v[i] = jnp.ones((16,), jnp.float32)
        return carry

    lax.fori_loop(0, RPT, _zero, 0)
    lax.fori_loop(0, EPW // 8, _one, 0)
    pltpu.sync_copy(zb_v, agg_sh.at[pl.ds(s * RPT, RPT)])
    pltpu.sync_copy(dst_hbm.at[wid], idx_v)
    plsc.subcore_barrier()
    for k in range(8):
        pltpu.sync_copy(ones_v, agg_sh.at[idx_v.at[k]], add=True)
    plsc.subcore_barrier()
    pltpu.sync_copy(agg_sh.at[pl.ds(s * RPT, RPT)],
                    out_hbm.at[c, pl.ds(s * RPT, RPT)])


@jax.jit
def _sc_indeg(dst38):
    return pl.kernel(
        _indeg_body,
        out_type=jax.ShapeDtypeStruct((2, NP, 16), jnp.float32),
        mesh=_mesh(),
        compiler_params=pltpu.CompilerParams(use_tc_tiling_on_sc=False),
        scratch_types=[
            pltpu.VMEM((8, EPW // 8), jnp.int32),
            pltpu.VMEM((EPW // 8, 16), jnp.float32),
            pltpu.VMEM((RPT, 16), jnp.float32),
            pltpu.VMEM_SHARED((NP, 16), jnp.float32),
        ],
    )(dst38)


@jax.jit
def _sc_scatter(msgs3, dst3):
    return pl.kernel(
        _scatter_body,
        out_type=jax.ShapeDtypeStruct((2, NP, 16), jnp.float32),
        mesh=_mesh(),
        compiler_params=pltpu.CompilerParams(use_tc_tiling_on_sc=False),
        scratch_types=[
            pltpu.VMEM((EPW,), jnp.int32),
            pltpu.VMEM((EPW, 16), jnp.float32),
            pltpu.VMEM((RPT, 16), jnp.float32),
            pltpu.SemaphoreType.DMA,
            pltpu.VMEM_SHARED((NP, 16), jnp.float32),
        ],
    )(msgs3, dst3)


_BR = 1024


def _unpack8(x8):
    t = x8.T
    return jnp.concatenate([t[16 * j:16 * j + 16, :] for j in range(8)], axis=1)


def _pack8(m, rows):
    t = jnp.concatenate([m[:, rows * j:rows * (j + 1)] for j in range(8)], axis=0)
    return t.T


def _mlp_body(xs_ref, xd_ref, w1a, w1b, b1, w2, b2, w3, b3, out_ref):
    bf = jnp.bfloat16
    f32 = jnp.float32
    xs = _unpack8(xs_ref[...]).astype(bf)
    xd = _unpack8(xd_ref[...]).astype(bf)
    h = (jnp.dot(w1a[...], xs, preferred_element_type=f32)
         + jnp.dot(w1b[...], xd, preferred_element_type=f32)).astype(bf)
    h = jax.nn.relu(h + b1[...])
    h = jax.nn.relu(jnp.dot(w2[...], h, preferred_element_type=f32).astype(bf)
                    + b2[...])
    m = jnp.dot(w3[...], h, preferred_element_type=f32) + b3[...]
    out_ref[...] = _pack8(m, _BR)


@jax.jit
def _tc_mlp(xs, xd, w1a, w1b, b1, w2, b2, w3, b3):
    n_blk = E8 // _BR
    full = lambda i: (0, 0)
    return pl.pallas_call(
        _mlp_body,
        grid=(n_blk,),
        in_specs=[
            pl.BlockSpec((_BR, 128), lambda i: (i, 0)),
            pl.BlockSpec((_BR, 128), lambda i: (i, 0)),
            pl.BlockSpec((MLP_H, 16), full),
            pl.BlockSpec((MLP_H, 16), full),
            pl.BlockSpec((MLP_H, 1), full),
            pl.BlockSpec((MLP_H, MLP_H), full),
            pl.BlockSpec((MLP_H, 1), full),
            pl.BlockSpec((16, MLP_H), full),
            pl.BlockSpec((16, 1), full),
        ],
        out_specs=pl.BlockSpec((_BR, 128), lambda i: (i, 0)),
        out_shape=jax.ShapeDtypeStruct((E8, 128), jnp.float32),
    )(xs, xd, w1a, w1b, b1, w2, b2, w3, b3)


def _update_body(agg2_ref, st_ref, ni_ref, wia, wib, whh, bih, bhh, wout, bout,
                 ns_ref, out_ref):
    f32 = jnp.float32
    agg = _unpack8(agg2_ref[0] + agg2_ref[1])
    st = _unpack8(st_ref[...])
    gx = (jnp.dot(wia[...], agg, preferred_element_type=f32)
          + jnp.dot(wib[...], ni_ref[...], preferred_element_type=f32)
          + bih[...])
    gh = jnp.dot(whh[...], st, preferred_element_type=f32) + bhh[...]
    r = jax.nn.sigmoid(gx[0:16] + gh[0:16])
    z = jax.nn.sigmoid(gx[16:32] + gh[16:32])
    n = jnp.tanh(gx[32:48] + r * gh[32:48])
    ns = (1.0 - z) * n + z * st
    ns_ref[...] = _pack8(ns, NP8)
    logits = jnp.dot(wout[...], ns, preferred_element_type=f32) + bout[...]
    row = lax.broadcasted_iota(jnp.int32, logits.shape, 0)
    logits = jnp.where(row < 9, logits, -1e30)
    m = jnp.max(logits, axis=0, keepdims=True)
    e = jnp.exp(logits - m)
    out_ref[...] = e / jnp.sum(e, axis=0, keepdims=True)


def _update1_body(ind2_ref, ni_ref, b1, w2, b2, w3, b3, wia, wib, bih, bhh,
                  wout, bout, ns_ref, out_ref):
    f32 = jnp.float32
    bf = jnp.bfloat16
    deg = _unpack8(ind2_ref[0] + ind2_ref[1])[0:1]
    h1 = jax.nn.relu(b1[...])
    h2 = jax.nn.relu(jnp.dot(w2[...], h1, preferred_element_type=f32).astype(bf)
                     + b2[...])
    m0 = jnp.dot(w3[...], h2, preferred_element_type=f32) + b3[...]
    agg = m0 * deg
    gx = (jnp.dot(wia[...], agg, preferred_element_type=f32)
          + jnp.dot(wib[...], ni_ref[...], preferred_element_type=f32)
          + bih[...])
    gh = jnp.broadcast_to(bhh[...], gx.shape)
    z = jax.nn.sigmoid(gx[16:32] + gh[16:32])
    r = jax.nn.sigmoid(gx[0:16] + gh[0:16])
    n = jnp.tanh(gx[32:48] + r * gh[32:48])
    ns = (1.0 - z) * n
    ns_ref[...] = _pack8(ns, NP8)
    logits = jnp.dot(wout[...], ns, preferred_element_type=f32) + bout[...]
    row = lax.broadcasted_iota(jnp.int32, logits.shape, 0)
    logits = jnp.where(row < 9, logits, -1e30)
    m = jnp.max(logits, axis=0, keepdims=True)
    e = jnp.exp(logits - m)
    out_ref[...] = e / jnp.sum(e, axis=0, keepdims=True)


@jax.jit
def _tc_update1(ind2, niT, b1c, w2m, b2c, w3m, b3c, wia, wib, bih, bhh,
                wout, bout):
    return pl.pallas_call(
        _update1_body,
        out_shape=[
            jax.ShapeDtypeStruct((NP8, 128), jnp.float32),
            jax.ShapeDtypeStruct((16, NP), jnp.float32),
        ],
    )(ind2, niT, b1c, w2m, b2c, w3m, b3c, wia, wib, bih, bhh, wout, bout)


@jax.jit
def _tc_update(agg2, state8, niT, wia, wib, whh, bih, bhh, wout, bout):
    return pl.pallas_call(
        _update_body,
        out_shape=[
            jax.ShapeDtypeStruct((NP8, 128), jnp.float32),
            jax.ShapeDtypeStruct((16, NP), jnp.float32),
        ],
    )(agg2, state8, niT, wia, wib, whh, bih, bhh, wout, bout)


def _prep_idx(ids):
    ids = ids.reshape(NW, N_EDGES // NW)
    pad = jnp.full((NW, EPW - N_EDGES // NW), SINK, dtype=jnp.int32)
    return jnp.concatenate([ids, pad], axis=1)


def _pad2(a, rows, cols):
    return jnp.zeros((rows, cols), a.dtype).at[: a.shape[0], : a.shape[1]].set(a)


def _gate_pad_rows(w, in_real):
    out = jnp.zeros((48, 16), w.dtype)
    for g in range(3):
        out = out.at[g * 16:g * 16 + DH, :in_real].set(w[g * DH:(g + 1) * DH])
    return out


def kernel(node_inputs, src_ids, dst_ids, W1, b1, W2, b2, W3, b3, Wout, bout,
           W_ih, W_hh, b_ih, b_hh):
    f32 = jnp.float32
    bf = jnp.bfloat16
    src3 = _prep_idx(src_ids)
    dst3 = _prep_idx(dst_ids)

    ni = _pad2(node_inputs.astype(f32), NP, 16)
    niT = ni.reshape(NP8, 8, 16).transpose(2, 1, 0).reshape(16, NP)

    w1a = _pad2(W1[:, :DH], MLP_H, 16).astype(bf)
    w1b = _pad2(W1[:, DH:], MLP_H, 16).astype(bf)
    b1c = b1.reshape(MLP_H, 1).astype(bf)
    w2m = W2.astype(bf)
    b2c = b2.reshape(MLP_H, 1).astype(bf)
    w3m = _pad2(W3, 16, MLP_H).astype(bf)
    b3c = _pad2(b3.reshape(DE, 1), 16, 1)

    wia = _gate_pad_rows(W_ih[:, :DE], DE)
    wib = _gate_pad_rows(W_ih[:, DE:], DIN)
    whh = _gate_pad_rows(W_hh, DH)
    bih = jnp.zeros((48, 1), f32)
    bhh = jnp.zeros((48, 1), f32)
    for g in range(3):
        bih = bih.at[g * 16:g * 16 + DH, 0].set(b_ih[g * DH:(g + 1) * DH])
        bhh = bhh.at[g * 16:g * 16 + DH, 0].set(b_hh[g * DH:(g + 1) * DH])
    woutp = _pad2(Wout, 16, 16)
    boutc = _pad2(bout.reshape(9, 1), 16, 1)

    ind2 = _sc_indeg(dst3.reshape(NW, 8, EPW // 8))
    state8, out_0 = _tc_update1(ind2.reshape(2, NP8, 128), niT, b1c, w2m, b2c,
                                w3m, b3c, wia, wib, bih, bhh, woutp, boutc)
    outs = [out_0]
    for _ in range(N_ITERS - 1):
        xs, xd = _sc_gather(state8.reshape(NP, 16), src3, dst3)
        msgs = _tc_mlp(xs.reshape(E8, 128), xd.reshape(E8, 128),
                       w1a, w1b, b1c, w2m, b2c, w3m, b3c)
        agg2 = _sc_scatter(msgs.reshape(NW, EPW, 16), dst3)
        state8, out_i = _tc_update(agg2.reshape(2, NP8, 128), state8, niT,
                                   wia, wib, whh, bih, bhh, woutp, boutc)
        outs.append(out_i)
    total = jnp.stack(outs, axis=0)
    total = total.reshape(N_ITERS, 16, 8, NP8).transpose(0, 3, 2, 1)
    return total.reshape(N_ITERS, NP, 16)[:, :N_NODES, :9]
